# baseline (speedup 1.0000x reference)
import jax
import jax.numpy as jnp
from jax.experimental import pallas as pl
from jax.experimental.pallas import tpu as pltpu

_K = 20
_A = 5
_BIAS_W = (1.08, 3.42, 6.63, 9.42, 16.62)
_BIAS_H = (1.19, 4.41, 11.38, 5.11, 10.52)
_THR = 0.45
_HW = 13
_CELLS = _HW * _HW
_N = _A * _CELLS
_NP = 896
_NS = 848
_KP = 24


def _yolo_nms_kernel(x_ref, im_ref, prob_ref, bbox_ref,
                     o_scr, sa_scr, sb_scr, sc_scr, scT_scr, cd_scr, cdT_scr,
                     res_scr):
    b = pl.program_id(0)
    im_h = im_ref[b, 0]
    im_w = im_ref[b, 1]
    xb = x_ref[0]

    ci = jax.lax.broadcasted_iota(jnp.int32, (1, _CELLS), 1)
    colf = (ci % _HW).astype(jnp.float32)
    rowf = (ci // _HW).astype(jnp.float32)

    sc_scr[...] = jnp.full((_KP, _NP), -1.0, jnp.float32)
    cd_scr[...] = jnp.zeros((8, _NP), jnp.float32)

    for a in range(_A):
        lo = a * _CELLS
        sl = pl.ds(lo, _CELLS)
        tx = xb[2 * a:2 * a + 1, :]
        ty = xb[2 * a + 1:2 * a + 2, :]
        tw = xb[10 + 2 * a:11 + 2 * a, :]
        th = xb[11 + 2 * a:12 + 2 * a, :]
        obj = jax.nn.sigmoid(xb[20 + a:21 + a, :])
        conf = xb[25 + _K * a:25 + _K * (a + 1), :]
        m = jnp.max(conf, axis=0, keepdims=True)
        e = jnp.exp(conf - m)
        p = e / jnp.sum(e, axis=0, keepdims=True)
        sc_scr[0:_K, sl] = p * obj

        bx = (jax.nn.sigmoid(tx) + colf) / 13.0
        by = (jax.nn.sigmoid(ty) + rowf) / 13.0
        bw = _BIAS_W[a] * jnp.exp(tw) / 13.0
        bh = _BIAS_H[a] * jnp.exp(th) / 13.0
        x1 = (bx - bw / 2.0) * im_w
        y1 = (by - bh / 2.0) * im_h
        x2 = (bx + bw / 2.0) * im_w
        y2 = (by + bh / 2.0) * im_h
        cd_scr[0:1, sl] = x1
        cd_scr[1:2, sl] = y1
        cd_scr[2:3, sl] = x2
        cd_scr[3:4, sl] = y2
        cd_scr[4:5, sl] = jnp.maximum(x2 - x1, 0.0) * jnp.maximum(y2 - y1, 0.0)

    s_all = sc_scr[...]
    sc_scr[...] = jnp.where(s_all > 0.0, s_all, 0.0)

    scT_scr[...] = sc_scr[...].T
    cdT_scr[...] = cd_scr[...].T

    x1c = cdT_scr[0:_NS, 0:1]
    y1c = cdT_scr[0:_NS, 1:2]
    x2c = cdT_scr[0:_NS, 2:3]
    y2c = cdT_scr[0:_NS, 3:4]
    arc = cdT_scr[0:_NS, 4:5]
    x1r = cd_scr[0:1, :]
    y1r = cd_scr[1:2, :]
    x2r = cd_scr[2:3, :]
    y2r = cd_scr[3:4, :]
    arr = cd_scr[4:5, :]
    ic = jax.lax.broadcasted_iota(jnp.int32, (_NS, 1), 0)
    ir = jax.lax.broadcasted_iota(jnp.int32, (1, _NP), 1)

    iw = jnp.maximum(jnp.minimum(x2c, x2r) - jnp.maximum(x1c, x1r), 0.0)
    ih = jnp.maximum(jnp.minimum(y2c, y2r) - jnp.maximum(y1c, y1r), 0.0)
    inter = iw * ih
    union = (arc + arr) - inter
    iou = inter / jnp.maximum(union, 1e-12)
    o_scr[...] = jnp.where((iou > _THR) & (ic != ir), 1.0, 0.0) \
        .astype(jnp.bfloat16)
    ksub = jax.lax.broadcasted_iota(jnp.int32, (_KP, _NP), 0)
    klane = jax.lax.broadcasted_iota(jnp.int32, (_NS, _KP), 1)

    def class_body(k, carry):
        srow = jnp.sum(jnp.where(ksub == k, sc_scr[...], 0.0),
                       axis=0, keepdims=True)
        scol = jnp.sum(jnp.where(klane == k, scT_scr[0:_NS, :], 0.0),
                       axis=1, keepdims=True)
        omat = o_scr[...]
        zero_b = jnp.zeros((), jnp.bfloat16)
        one_b = jnp.ones((), jnp.bfloat16)

        def cond(c):
            return c[2]

        def run(sa_mask_f32):
            sa = omat * sa_mask_f32.astype(jnp.bfloat16)
            sa_scr[...] = sa
            sb_scr[...] = omat - sa

            def dstep(keep):
                supc = jnp.max(sb_scr[...] * keep, axis=1, keepdims=True)
                keep_col = jnp.where(supc > zero_b, zero_b, one_b)
                supr = jnp.max(sa_scr[...] * keep_col, axis=0, keepdims=True)
                return jnp.where(supr > zero_b, zero_b, one_b)

            def body(c):
                keep, ssum, _ = c
                keep2 = dstep(keep)
                s2 = jnp.sum(keep2.astype(jnp.float32))
                return keep2, s2, ssum > s2

            supc0 = jnp.max(sb_scr[...], axis=1, keepdims=True)
            keep_col0 = jnp.where(supc0 > zero_b, zero_b, one_b)
            supr0 = jnp.max(sa_scr[...] * keep_col0, axis=0, keepdims=True)
            keep = jnp.where(supr0 > zero_b, zero_b, one_b)
            for _ in range(1):
                keep = dstep(keep)
            s0 = jnp.sum(keep.astype(jnp.float32))
            keep_f, _, _ = jax.lax.while_loop(
                cond, body, (keep, s0, jnp.bool_(True)))
            return keep_f

        ca = (scol > srow) | ((scol == srow) & (ic < ir))
        keep_f = run(jnp.where(ca, 1.0, 0.0))
        res_scr[...] = jnp.where(ksub == k,
                                 keep_f.astype(jnp.float32) * srow,
                                 res_scr[...])
        return carry

    jax.lax.fori_loop(0, _K, class_body, 0)
    prob_ref[0] = res_scr[0:_K, :]
    bbox_ref[0] = cd_scr[...]


def kernel(x, im_info):
    B = x.shape[0]
    x2 = x.reshape(B, 125, _CELLS)
    prob_pad, bbox_pad = pl.pallas_call(
        _yolo_nms_kernel,
        grid=(B,),
        in_specs=[
            pl.BlockSpec((1, 125, _CELLS), lambda b: (b, 0, 0)),
            pl.BlockSpec(memory_space=pltpu.SMEM),
        ],
        out_specs=[
            pl.BlockSpec((1, _K, _NP), lambda b: (b, 0, 0)),
            pl.BlockSpec((1, 8, _NP), lambda b: (b, 0, 0)),
        ],
        out_shape=[
            jax.ShapeDtypeStruct((B, _K, _NP), jnp.float32),
            jax.ShapeDtypeStruct((B, 8, _NP), jnp.float32),
        ],
        scratch_shapes=[
            pltpu.VMEM((_NS, _NP), jnp.bfloat16),
            pltpu.VMEM((_NS, _NP), jnp.bfloat16),
            pltpu.VMEM((_NS, _NP), jnp.bfloat16),
            pltpu.VMEM((_KP, _NP), jnp.float32),
            pltpu.VMEM((_NP, _KP), jnp.float32),
            pltpu.VMEM((8, _NP), jnp.float32),
            pltpu.VMEM((_NP, 8), jnp.float32),
            pltpu.VMEM((_KP, _NP), jnp.float32),
        ],
    )(x2, im_info)
    prob = prob_pad[:, :, :_N].transpose(0, 2, 1)
    bboxs = bbox_pad[:, :4, :_N].transpose(0, 2, 1)
    return (prob, bboxs)


# speedup vs baseline: 1.0468x; 1.0468x over previous
import jax
import jax.numpy as jnp
from jax.experimental import pallas as pl
from jax.experimental.pallas import tpu as pltpu

_K = 20
_A = 5
_BIAS_W = (1.08, 3.42, 6.63, 9.42, 16.62)
_BIAS_H = (1.19, 4.41, 11.38, 5.11, 10.52)
_THR = 0.45
_HW = 13
_CELLS = _HW * _HW
_N = _A * _CELLS
_NP = 896
_NS = 848
_KP = 24


def _yolo_nms_kernel(x_ref, im_ref, prob_ref, bbox_ref,
                     o_scr, sa_scr, sb_scr, sc_scr, scT_scr, cd_scr, cdT_scr,
                     res_scr):
    b = pl.program_id(0)
    im_h = im_ref[b, 0]
    im_w = im_ref[b, 1]
    xb = x_ref[0]

    ci = jax.lax.broadcasted_iota(jnp.int32, (1, _CELLS), 1)
    colf = (ci % _HW).astype(jnp.float32)
    rowf = (ci // _HW).astype(jnp.float32)

    sc_scr[...] = jnp.full((_KP, _NP), -1.0, jnp.float32)
    cd_scr[...] = jnp.zeros((8, _NP), jnp.float32)

    for a in range(_A):
        lo = a * _CELLS
        sl = pl.ds(lo, _CELLS)
        tx = xb[2 * a:2 * a + 1, :]
        ty = xb[2 * a + 1:2 * a + 2, :]
        tw = xb[10 + 2 * a:11 + 2 * a, :]
        th = xb[11 + 2 * a:12 + 2 * a, :]
        obj = jax.nn.sigmoid(xb[20 + a:21 + a, :])
        conf = xb[25 + _K * a:25 + _K * (a + 1), :]
        m = jnp.max(conf, axis=0, keepdims=True)
        e = jnp.exp(conf - m)
        p = e / jnp.sum(e, axis=0, keepdims=True)
        sc_scr[0:_K, sl] = p * obj

        bx = (jax.nn.sigmoid(tx) + colf) / 13.0
        by = (jax.nn.sigmoid(ty) + rowf) / 13.0
        bw = _BIAS_W[a] * jnp.exp(tw) / 13.0
        bh = _BIAS_H[a] * jnp.exp(th) / 13.0
        x1 = (bx - bw / 2.0) * im_w
        y1 = (by - bh / 2.0) * im_h
        x2 = (bx + bw / 2.0) * im_w
        y2 = (by + bh / 2.0) * im_h
        cd_scr[0:1, sl] = x1
        cd_scr[1:2, sl] = y1
        cd_scr[2:3, sl] = x2
        cd_scr[3:4, sl] = y2
        cd_scr[4:5, sl] = jnp.maximum(x2 - x1, 0.0) * jnp.maximum(y2 - y1, 0.0)

    s_all = sc_scr[...]
    sc_scr[...] = jnp.where(s_all > 0.0, s_all, 0.0)

    scT_scr[...] = sc_scr[...].T
    cdT_scr[...] = cd_scr[...].T

    x1c = cdT_scr[0:_NS, 0:1]
    y1c = cdT_scr[0:_NS, 1:2]
    x2c = cdT_scr[0:_NS, 2:3]
    y2c = cdT_scr[0:_NS, 3:4]
    arc = cdT_scr[0:_NS, 4:5]
    x1r = cd_scr[0:1, :]
    y1r = cd_scr[1:2, :]
    x2r = cd_scr[2:3, :]
    y2r = cd_scr[3:4, :]
    arr = cd_scr[4:5, :]
    ic = jax.lax.broadcasted_iota(jnp.int32, (_NS, 1), 0)
    ir = jax.lax.broadcasted_iota(jnp.int32, (1, _NP), 1)

    iw = jnp.maximum(jnp.minimum(x2c, x2r) - jnp.maximum(x1c, x1r), 0.0)
    ih = jnp.maximum(jnp.minimum(y2c, y2r) - jnp.maximum(y1c, y1r), 0.0)
    inter = iw * ih
    union = (arc + arr) - inter
    iou = inter / jnp.maximum(union, 1e-12)
    o_scr[...] = jnp.where((iou > _THR) & (ic != ir), 1.0, 0.0) \
        .astype(jnp.bfloat16)
    ksub = jax.lax.broadcasted_iota(jnp.int32, (_KP, _NP), 0)
    klane = jax.lax.broadcasted_iota(jnp.int32, (_NS, _KP), 1)

    def class_body(k, carry):
        srow = jnp.sum(jnp.where(ksub == k, sc_scr[...], 0.0),
                       axis=0, keepdims=True)
        scol = jnp.sum(jnp.where(klane == k, scT_scr[0:_NS, :], 0.0),
                       axis=1, keepdims=True)
        omat = o_scr[...]
        zero_b = jnp.zeros((), jnp.bfloat16)
        one_b = jnp.ones((), jnp.bfloat16)

        def cond(c):
            return c[2]

        def run(sa_mask_f32):
            sa = omat * sa_mask_f32.astype(jnp.bfloat16)
            sa_scr[...] = sa
            sb_scr[...] = omat - sa

            def dstep(keep):
                supc = jnp.max(sb_scr[...] * keep, axis=1, keepdims=True)
                keep_col = jnp.where(supc > zero_b, zero_b, one_b)
                supr = jnp.max(sa_scr[...] * keep_col, axis=0, keepdims=True)
                return jnp.where(supr > zero_b, zero_b, one_b)

            def body(c):
                keep, ssum, _ = c
                keep2 = dstep(keep)
                s2 = jnp.sum(keep2.astype(jnp.float32))
                return keep2, s2, ssum > s2

            supc0 = jnp.max(sb_scr[...], axis=1, keepdims=True)
            keep_col0 = jnp.where(supc0 > zero_b, zero_b, one_b)
            supr0 = jnp.max(sa_scr[...] * keep_col0, axis=0, keepdims=True)
            keep = jnp.where(supr0 > zero_b, zero_b, one_b)
            for _ in range(2):
                keep = dstep(keep)
            s0 = jnp.sum(keep.astype(jnp.float32))
            keep_f, _, _ = jax.lax.while_loop(
                cond, body, (keep, s0, jnp.bool_(True)))
            return keep_f

        ca = (scol > srow) | ((scol == srow) & (ic < ir))
        keep_f = run(jnp.where(ca, 1.0, 0.0))
        res_scr[...] = jnp.where(ksub == k,
                                 keep_f.astype(jnp.float32) * srow,
                                 res_scr[...])
        return carry

    jax.lax.fori_loop(0, _K, class_body, 0)
    prob_ref[0] = res_scr[0:_K, :]
    bbox_ref[0] = cd_scr[...]


def kernel(x, im_info):
    B = x.shape[0]
    x2 = x.reshape(B, 125, _CELLS)
    prob_pad, bbox_pad = pl.pallas_call(
        _yolo_nms_kernel,
        grid=(B,),
        in_specs=[
            pl.BlockSpec((1, 125, _CELLS), lambda b: (b, 0, 0)),
            pl.BlockSpec(memory_space=pltpu.SMEM),
        ],
        out_specs=[
            pl.BlockSpec((1, _K, _NP), lambda b: (b, 0, 0)),
            pl.BlockSpec((1, 8, _NP), lambda b: (b, 0, 0)),
        ],
        out_shape=[
            jax.ShapeDtypeStruct((B, _K, _NP), jnp.float32),
            jax.ShapeDtypeStruct((B, 8, _NP), jnp.float32),
        ],
        scratch_shapes=[
            pltpu.VMEM((_NS, _NP), jnp.bfloat16),
            pltpu.VMEM((_NS, _NP), jnp.bfloat16),
            pltpu.VMEM((_NS, _NP), jnp.bfloat16),
            pltpu.VMEM((_KP, _NP), jnp.float32),
            pltpu.VMEM((_NP, _KP), jnp.float32),
            pltpu.VMEM((8, _NP), jnp.float32),
            pltpu.VMEM((_NP, 8), jnp.float32),
            pltpu.VMEM((_KP, _NP), jnp.float32),
        ],
    )(x2, im_info)
    prob = prob_pad[:, :, :_N].transpose(0, 2, 1)
    bboxs = bbox_pad[:, :4, :_N].transpose(0, 2, 1)
    return (prob, bboxs)
